# Initial kernel scaffold; baseline (speedup 1.0000x reference)
import jax
import jax.numpy as jnp
from jax.experimental import pallas as pl


def kernel(x, W_enc, b_enc, W_dec, b_dec):
    raise NotImplementedError("write your pallas kernel here")


# speedup vs baseline: 6.7868x; 6.7868x over previous
import functools

import jax
import jax.numpy as jnp
from jax import lax
from jax.experimental import pallas as pl

_K = 32


def _enc_body(x_ref, wt_ref, b_ref, z_ref):
    z = (
        lax.dot_general(
            x_ref[...],
            wt_ref[...],
            (((1,), (1,)), ((), ())),
            preferred_element_type=jnp.float32,
        )
        + b_ref[...]
    )
    bits = lax.bitcast_convert_type(z, jnp.int32) & 0x7FFFFFFF
    rb = z.shape[0]

    def body(_, carry):
        lo, hi = carry
        mid = lo + ((hi - lo) >> 1)
        cnt = jnp.sum((bits >= mid).astype(jnp.int32), axis=1, keepdims=True)
        ge = cnt >= _K
        return jnp.where(ge, mid, lo), jnp.where(ge, hi, mid)

    lo0 = jnp.zeros((rb, 1), jnp.int32)
    hi0 = jnp.full((rb, 1), 0x7F800000, jnp.int32)
    lo, _ = lax.fori_loop(0, 31, body, (lo0, hi0))
    z_ref[...] = jnp.where(bits >= lo, z, 0.0)


def _dec_body(z_ref, wt_ref, b_ref, o_ref):
    o_ref[...] = (
        lax.dot_general(
            z_ref[...],
            wt_ref[...],
            (((1,), (1,)), ((), ())),
            preferred_element_type=jnp.float32,
        )
        + b_ref[...]
    )


@jax.jit
def kernel(x, W_enc, b_enc, W_dec, b_dec):
    n, d_model = x.shape
    d_dict = W_enc.shape[0]
    rb = 128
    grid = (n // rb,)

    z = pl.pallas_call(
        _enc_body,
        grid=grid,
        in_specs=[
            pl.BlockSpec((rb, d_model), lambda i: (i, 0)),
            pl.BlockSpec((d_dict, d_model), lambda i: (0, 0)),
            pl.BlockSpec((1, d_dict), lambda i: (0, 0)),
        ],
        out_specs=pl.BlockSpec((rb, d_dict), lambda i: (i, 0)),
        out_shape=jax.ShapeDtypeStruct((n, d_dict), jnp.float32),
    )(x, W_enc, b_enc.reshape(1, d_dict))

    out = pl.pallas_call(
        _dec_body,
        grid=grid,
        in_specs=[
            pl.BlockSpec((rb, d_dict), lambda i: (i, 0)),
            pl.BlockSpec((d_model, d_dict), lambda i: (0, 0)),
            pl.BlockSpec((1, d_model), lambda i: (0, 0)),
        ],
        out_specs=pl.BlockSpec((rb, d_model), lambda i: (i, 0)),
        out_shape=jax.ShapeDtypeStruct((n, d_model), jnp.float32),
    )(z, W_dec, b_dec.reshape(1, d_model))

    return (out, z)
